# baseline (speedup 1.0000x reference)
import functools

import jax
import jax.numpy as jnp
from jax import lax
from jax.experimental import pallas as pl
from jax.experimental.pallas import tpu as pltpu
from jax.experimental.pallas import tpu_sc as plsc


def _round_up(x: int, m: int) -> int:
    return (x + m - 1) // m * m


@functools.lru_cache(maxsize=None)
def _make_gather(b: int, l: int, lp: int, d: int):
    info = plsc.get_sparse_core_info()
    nc, ns = info.num_cores, info.num_subcores
    nw = nc * ns
    rows_per_w = b // nw
    assert rows_per_w * nw == b and rows_per_w % 8 == 0
    mesh = plsc.VectorSubcoreMesh(core_axis_name="c", subcore_axis_name="s")

    @functools.partial(
        pl.kernel,
        mesh=mesh,
        out_type=jax.ShapeDtypeStruct((b, lp, d), jnp.float32),
        scratch_types=[
            pltpu.VMEM((rows_per_w, lp), jnp.int32),
            pltpu.VMEM((lp, d), jnp.float32),
            pltpu.SemaphoreType.DMA,
        ],
        compiler_params=pltpu.CompilerParams(use_tc_tiling_on_sc=True),
    )
    def gather(table_hbm, idx_hbm, out_hbm, idx_v, rows_v, sem):
        wid = lax.axis_index("s") * nc + lax.axis_index("c")
        base = wid * rows_per_w
        pltpu.sync_copy(idx_hbm.at[pl.ds(base, rows_per_w)], idx_v)

        def body(j, carry):
            pltpu.async_copy(table_hbm.at[idx_v.at[j]], rows_v, sem).wait()
            pltpu.sync_copy(rows_v, out_hbm.at[base + j])
            return carry

        lax.fori_loop(0, rows_per_w, body, 0)

    return gather


def _mlp_body(l, emb_ref, w1_ref, b1_ref, w2_ref, b2_ref, out_ref):
    g, lp, d = emb_ref.shape
    emb = emb_ref[...].reshape(g * lp, d)
    h = jnp.dot(emb, w1_ref[...], preferred_element_type=jnp.float32)
    h = jnp.maximum(h + b1_ref[...], 0.0)
    h3 = h.reshape(g, lp, -1)
    hc = jnp.concatenate([h3[i, :l, :] for i in range(g)], axis=0)
    out_ref[...] = (
        jnp.dot(hc, w2_ref[...], preferred_element_type=jnp.float32)
        + b2_ref[...]
    )


@functools.lru_cache(maxsize=None)
def _make_mlp(b: int, l: int, lp: int, d: int, hidden: int, vocab: int, g: int):
    grid = (b // g,)
    return pl.pallas_call(
        functools.partial(_mlp_body, l),
        grid=grid,
        in_specs=[
            pl.BlockSpec((g, lp, d), lambda i: (i, 0, 0)),
            pl.BlockSpec((d, hidden), lambda i: (0, 0)),
            pl.BlockSpec((1, hidden), lambda i: (0, 0)),
            pl.BlockSpec((hidden, vocab), lambda i: (0, 0)),
            pl.BlockSpec((1, vocab), lambda i: (0, 0)),
        ],
        out_specs=pl.BlockSpec((g * l, vocab), lambda i: (i, 0)),
        out_shape=jax.ShapeDtypeStruct((b * l, vocab), jnp.float32),
        compiler_params=pltpu.CompilerParams(
            dimension_semantics=("parallel",),
        ),
    )


def kernel(input_ids, table, W1, b1, W2, b2):
    b, l = input_ids.shape
    vocab, d = table.shape
    hidden = W1.shape[1]
    lp = _round_up(l, 8)

    ids = jnp.pad(input_ids.astype(jnp.int32), ((0, 0), (0, lp - l)), mode="edge")
    emb = _make_gather(b, l, lp, d)(table, ids)

    logits = _make_mlp(b, l, lp, d, hidden, vocab, 8)(
        emb, W1, b1.reshape(1, hidden), W2, b2.reshape(1, vocab)
    )
    return logits.reshape(b, l, vocab)


# speedup vs baseline: 3.5175x; 3.5175x over previous
import functools

import jax
import jax.numpy as jnp
from jax import lax
from jax.experimental import pallas as pl
from jax.experimental.pallas import tpu as pltpu
from jax.experimental.pallas import tpu_sc as plsc


@functools.lru_cache(maxsize=None)
def _make_gather(b: int, l: int, d: int, bc: int):
    info = plsc.get_sparse_core_info()
    nc, ns = info.num_cores, info.num_subcores
    nw = nc * ns
    nbc = b // bc
    units = l * nbc
    iters = (units + nw - 1) // nw
    mesh = plsc.VectorSubcoreMesh(core_axis_name="c", subcore_axis_name="s")

    @functools.partial(
        pl.kernel,
        mesh=mesh,
        out_type=jax.ShapeDtypeStruct((l, b, d), jnp.float32),
        scratch_types=[
            pltpu.VMEM((l, nbc, bc), jnp.int32),
            pltpu.VMEM((bc, d), jnp.float32),
            pltpu.SemaphoreType.DMA,
        ],
        compiler_params=pltpu.CompilerParams(use_tc_tiling_on_sc=True),
    )
    def gather(table_hbm, idx_hbm, out_hbm, idx_v, rows_v, sem):
        wid = lax.axis_index("s") * nc + lax.axis_index("c")
        pltpu.sync_copy(idx_hbm, idx_v)

        def body(k, carry):
            g = wid + nw * k

            @pl.when(g < units)
            def _():
                li = g // nbc
                ci = g % nbc
                pltpu.async_copy(
                    table_hbm.at[idx_v.at[li].at[ci]], rows_v, sem
                ).wait()
                pltpu.sync_copy(rows_v, out_hbm.at[li].at[pl.ds(ci * bc, bc)])

            return carry

        lax.fori_loop(0, iters, body, 0)

    return gather


def _mlp_body(emb_ref, w1t_ref, b1_ref, w2t_ref, b2_ref, out_ref):
    embt = emb_ref[0].T
    h = jnp.dot(w1t_ref[...], embt, preferred_element_type=jnp.float32)
    h = jnp.maximum(h + b1_ref[...], 0.0)
    out_ref[0] = (
        jnp.dot(w2t_ref[...], h, preferred_element_type=jnp.float32)
        + b2_ref[...]
    )


@functools.lru_cache(maxsize=None)
def _make_mlp(b: int, l: int, d: int, hidden: int, vocab: int):
    return pl.pallas_call(
        _mlp_body,
        grid=(l,),
        in_specs=[
            pl.BlockSpec((1, b, d), lambda i: (i, 0, 0)),
            pl.BlockSpec((hidden, d), lambda i: (0, 0)),
            pl.BlockSpec((hidden, 1), lambda i: (0, 0)),
            pl.BlockSpec((vocab, hidden), lambda i: (0, 0)),
            pl.BlockSpec((vocab, 1), lambda i: (0, 0)),
        ],
        out_specs=pl.BlockSpec((1, vocab, b), lambda i: (i, 0, 0)),
        out_shape=jax.ShapeDtypeStruct((l, vocab, b), jnp.float32),
        compiler_params=pltpu.CompilerParams(
            dimension_semantics=("parallel",),
        ),
    )


def kernel(input_ids, table, W1, b1, W2, b2):
    b, l = input_ids.shape
    vocab, d = table.shape
    hidden = W1.shape[1]
    bc = 128

    idsT3 = jnp.clip(
        input_ids.astype(jnp.int32).T.reshape(l, b // bc, bc), 0, vocab - 1
    )
    embT = _make_gather(b, l, d, bc)(table, idsT3)

    logitsT = _make_mlp(b, l, d, hidden, vocab)(
        embT,
        W1.T,
        b1.reshape(hidden, 1),
        W2.T,
        b2.reshape(vocab, 1),
    )
    return jnp.transpose(logitsT, (2, 0, 1))
